# baseline (speedup 1.0000x reference)
import jax
import jax.numpy as jnp
from jax.experimental import pallas as pl

N_ANCHORS = 20000
BATCH = 8
L = 101
CHUNK = 4096


def _encode_tc_kernel(bt_ref, btt_ref, db_ref, lab_ref, *iou_refs):
    db = db_ref[...]
    db_x0 = db[0:1, :]
    db_y0 = db[1:2, :]
    db_x1 = db[2:3, :]
    db_y1 = db[3:4, :]
    p_w = db_x1 - db_x0
    p_h = db_y1 - db_y0
    area2 = p_w * p_h
    p_cx = (db_x1 + db_x0) * 0.5
    p_cy = (db_y1 + db_y0) * 0.5
    lane = jax.lax.broadcasted_iota(jnp.int32, (1, db.shape[1]), 1)
    valid_lane = (lane + pl.program_id(0) * CHUNK) < N_ANCHORS

    def emit(b, pos, sel5):
        cls_raw = jnp.where(pos, sel5[4:5, :], 0.0)
        bg = cls_raw == 0.0
        tx0 = jnp.where(bg, db_x0, sel5[0:1, :])
        ty0 = jnp.where(bg, db_y0, sel5[1:2, :])
        tx1 = jnp.where(bg, db_x1, sel5[2:3, :])
        ty1 = jnp.where(bg, db_y1, sel5[3:4, :])
        t_cx = ((tx1 + tx0) * 0.5 - p_cx) / p_w
        t_cy = ((ty1 + ty0) * 0.5 - p_cy) / p_h
        t_w = jnp.log((tx1 - tx0) / p_w)
        t_h = jnp.log((ty1 - ty0) / p_h)
        lab_ref[b] = jnp.concatenate(
            [t_cx, t_cy, t_w, t_h, cls_raw], axis=0)

    ties = []
    for b in range(BATCH):
        bt = bt_ref[b]
        bx0 = bt[:, 0:1]
        by0 = bt[:, 1:2]
        bx1 = bt[:, 2:3]
        by1 = bt[:, 3:4]
        iw = jnp.maximum(jnp.minimum(bx1, db_x1) - jnp.maximum(bx0, db_x0), 0.0)
        ih = jnp.maximum(jnp.minimum(by1, db_y1) - jnp.maximum(by0, db_y0), 0.0)
        inter = iw * ih
        area1 = (bx1 - bx0) * (by1 - by0)
        union = area1 + area2 - inter
        r = pl.reciprocal(union, approx=True)
        r = r * (2.0 - union * r)
        iou = inter * r
        iou_refs[b][...] = iou
        iou_max = jnp.max(iou, axis=0, keepdims=True)
        pos = iou_max > 0.5

        hit = (iou == iou_max).astype(jnp.bfloat16)
        sel2 = jax.lax.dot_general(
            btt_ref[b], hit,
            dimension_numbers=(((1,), (0,)), ((), ())),
            preferred_element_type=jnp.float32)
        cnt = sel2[10:11, :]
        emit(b, pos, sel2[0:5, :] + sel2[5:10, :])
        ties.append(jnp.max(jnp.where(pos & valid_lane, cnt, 0.0)) > 1.5)

    tie_any = ties[0]
    for t in ties[1:]:
        tie_any = tie_any | t

    @pl.when(tie_any)
    def _exact_tiebreak():
        for b in range(BATCH):
            iou = iou_refs[b][...]
            iou_max = jnp.max(iou, axis=0, keepdims=True)
            pos = iou_max > 0.5
            row_iota = jax.lax.broadcasted_iota(jnp.int32, iou.shape, 0)
            idx = jnp.min(jnp.where(iou >= iou_max, row_iota, L),
                          axis=0, keepdims=True)
            onehot = (row_iota == idx).astype(jnp.bfloat16)
            sel2x = jax.lax.dot_general(
                btt_ref[b][:10], onehot,
                dimension_numbers=(((1,), (0,)), ((), ())),
                preferred_element_type=jnp.float32)
            emit(b, pos, sel2x[0:5, :] + sel2x[5:10, :])


def kernel(labels, default_boxes):
    valid = (labels[..., 4:5] != 0).astype(labels.dtype)
    masked = labels * valid
    box_true = jnp.concatenate(
        [jnp.zeros((BATCH, 1, 5), labels.dtype), masked], axis=1)
    box_true_t = box_true.transpose(0, 2, 1)
    btt_hi = box_true_t.astype(jnp.bfloat16)
    btt_lo = (box_true_t - btt_hi.astype(jnp.float32)).astype(jnp.bfloat16)
    ones_row = jnp.ones((BATCH, 1, L), jnp.bfloat16)
    btt2 = jnp.concatenate([btt_hi, btt_lo, ones_row], axis=1)
    db_t = default_boxes.T
    n_chunks = pl.cdiv(N_ANCHORS, CHUNK)
    out_shape = (
        [jax.ShapeDtypeStruct((BATCH, 5, N_ANCHORS), jnp.float32)]
        + [jax.ShapeDtypeStruct((L, N_ANCHORS), jnp.float32)
           for _ in range(BATCH)]
    )
    outs = pl.pallas_call(
        _encode_tc_kernel,
        grid=(n_chunks,),
        in_specs=[
            pl.BlockSpec((BATCH, L, 5), lambda i: (0, 0, 0)),
            pl.BlockSpec((BATCH, 11, L), lambda i: (0, 0, 0)),
            pl.BlockSpec((4, CHUNK), lambda i: (0, i)),
        ],
        out_specs=(
            [pl.BlockSpec((BATCH, 5, CHUNK), lambda i: (0, 0, i))]
            + [pl.BlockSpec((L, CHUNK), lambda i: (0, i))
               for _ in range(BATCH)]
        ),
        out_shape=out_shape,
    )(box_true, btt2, db_t)
    labeled = outs[0].transpose(0, 2, 1)
    return labeled, tuple(outs[1:])


# speedup vs baseline: 1.1234x; 1.1234x over previous
import jax
import jax.numpy as jnp
from jax.experimental import pallas as pl

N_ANCHORS = 20000
BATCH = 8
MAX_LABELS = 100
L = 101
CHUNK = 4096


def _encode_tc_kernel(labels_ref, db_ref, lab_ref, *iou_refs):
    db = db_ref[...]
    db_x0 = db[0:1, :]
    db_y0 = db[1:2, :]
    db_x1 = db[2:3, :]
    db_y1 = db[3:4, :]
    p_w = db_x1 - db_x0
    p_h = db_y1 - db_y0
    area2 = p_w * p_h
    p_cx = (db_x1 + db_x0) * 0.5
    p_cy = (db_y1 + db_y0) * 0.5
    lane = jax.lax.broadcasted_iota(jnp.int32, (1, db.shape[1]), 1)
    valid_lane = (lane + pl.program_id(0) * CHUNK) < N_ANCHORS

    def make_table(b):
        lab = labels_ref[b]
        valid = (lab[:, 4:5] != 0.0).astype(jnp.float32)
        bt = jnp.concatenate(
            [jnp.zeros((1, 5), jnp.float32), lab * valid], axis=0)
        bt_hi = bt.astype(jnp.bfloat16)
        bt_lo = (bt - bt_hi.astype(jnp.float32)).astype(jnp.bfloat16)
        ones_col = jnp.ones((L, 1), jnp.bfloat16)
        return bt, jnp.concatenate([bt_hi, bt_lo, ones_col], axis=1)

    def emit(b, pos, sel5):
        cls_raw = jnp.where(pos, sel5[4:5, :], 0.0)
        bg = cls_raw == 0.0
        tx0 = jnp.where(bg, db_x0, sel5[0:1, :])
        ty0 = jnp.where(bg, db_y0, sel5[1:2, :])
        tx1 = jnp.where(bg, db_x1, sel5[2:3, :])
        ty1 = jnp.where(bg, db_y1, sel5[3:4, :])
        t_cx = ((tx1 + tx0) * 0.5 - p_cx) / p_w
        t_cy = ((ty1 + ty0) * 0.5 - p_cy) / p_h
        t_w = jnp.log((tx1 - tx0) / p_w)
        t_h = jnp.log((ty1 - ty0) / p_h)
        lab_ref[b] = jnp.concatenate(
            [t_cx, t_cy, t_w, t_h, cls_raw], axis=0)

    ties = []
    for b in range(BATCH):
        bt, bt2 = make_table(b)
        bx0 = bt[:, 0:1]
        by0 = bt[:, 1:2]
        bx1 = bt[:, 2:3]
        by1 = bt[:, 3:4]
        iw = jnp.maximum(jnp.minimum(bx1, db_x1) - jnp.maximum(bx0, db_x0), 0.0)
        ih = jnp.maximum(jnp.minimum(by1, db_y1) - jnp.maximum(by0, db_y0), 0.0)
        inter = iw * ih
        area1 = (bx1 - bx0) * (by1 - by0)
        iou = inter / (area1 + area2 - inter)
        iou_refs[b][...] = iou
        iou_max = jnp.max(iou, axis=0, keepdims=True)
        pos = iou_max > 0.5
        hit = (iou == iou_max).astype(jnp.bfloat16)
        sel2 = jax.lax.dot_general(
            bt2, hit,
            dimension_numbers=(((0,), (0,)), ((), ())),
            preferred_element_type=jnp.float32)
        cnt = sel2[10:11, :]
        emit(b, pos, sel2[0:5, :] + sel2[5:10, :])
        ties.append(jnp.max(jnp.where(pos & valid_lane, cnt, 0.0)) > 1.5)

    tie_any = ties[0]
    for t in ties[1:]:
        tie_any = tie_any | t

    @pl.when(tie_any)
    def _exact_tiebreak():
        for b in range(BATCH):
            _, bt2 = make_table(b)
            iou = iou_refs[b][...]
            iou_max = jnp.max(iou, axis=0, keepdims=True)
            pos = iou_max > 0.5
            row_iota = jax.lax.broadcasted_iota(jnp.int32, iou.shape, 0)
            idx = jnp.min(jnp.where(iou >= iou_max, row_iota, L),
                          axis=0, keepdims=True)
            onehot = (row_iota == idx).astype(jnp.bfloat16)
            sel2x = jax.lax.dot_general(
                bt2[:, :10], onehot,
                dimension_numbers=(((0,), (0,)), ((), ())),
                preferred_element_type=jnp.float32)
            emit(b, pos, sel2x[0:5, :] + sel2x[5:10, :])


def kernel(labels, default_boxes):
    db_t = default_boxes.T
    n_chunks = pl.cdiv(N_ANCHORS, CHUNK)
    out_shape = (
        [jax.ShapeDtypeStruct((BATCH, 5, N_ANCHORS), jnp.float32)]
        + [jax.ShapeDtypeStruct((L, N_ANCHORS), jnp.float32)
           for _ in range(BATCH)]
    )
    outs = pl.pallas_call(
        _encode_tc_kernel,
        grid=(n_chunks,),
        in_specs=[
            pl.BlockSpec((BATCH, MAX_LABELS, 5), lambda i: (0, 0, 0)),
            pl.BlockSpec((4, CHUNK), lambda i: (0, i)),
        ],
        out_specs=(
            [pl.BlockSpec((BATCH, 5, CHUNK), lambda i: (0, 0, i))]
            + [pl.BlockSpec((L, CHUNK), lambda i: (0, i))
               for _ in range(BATCH)]
        ),
        out_shape=out_shape,
    )(labels, db_t)
    labeled = outs[0].transpose(0, 2, 1)
    return labeled, tuple(outs[1:])
